# baseline (speedup 1.0000x reference)
import functools

import numpy as np
import jax
import jax.numpy as jnp
from jax import lax
from jax.experimental import pallas as pl
from jax.experimental.pallas import tpu as pltpu
from jax.experimental.pallas import tpu_sc as plsc

_NUM_HEADS = 16
_ATTN_HEIGHT = 24
_CLS_TOKEN_LEN = 1
_CLS_TOKEN_POS_LEN = 3

_HG = 4
_NG = 4
_S = 577
_PR = 592
_PC = 640
_TR = 74
_TRU = 8 * _PC
_NRD = 2212


def _build_rpi_perm():
    height = _ATTN_HEIGHT
    width = (_S - _CLS_TOKEN_LEN) // height
    hh, ww = np.meshgrid(range(height), range(width))
    coords = np.stack([hh, ww], axis=-1)
    coords_flatten = np.reshape(coords, [-1, 2])
    relative_coords = coords_flatten[:, None, :] - coords_flatten[None, :, :]
    relative_coords_hh = relative_coords[:, :, 0] + height - 1
    relative_coords_ww = (relative_coords[:, :, 1] + width - 1) * (2 * height - 1)
    relative_coords = np.stack([relative_coords_hh, relative_coords_ww], axis=-1)
    rpi = np.sum(relative_coords, axis=-1).astype(np.int64)
    num_relative_distance = (2 * height - 1) * (2 * width - 1) + _CLS_TOKEN_POS_LEN
    top = np.full((1, rpi.shape[1]), num_relative_distance - 3, dtype=rpi.dtype)
    left = np.full((rpi.shape[0], 1), num_relative_distance - 2, dtype=rpi.dtype)
    corner = np.full((1, 1), num_relative_distance - 1, dtype=rpi.dtype)
    left_corner = np.concatenate([corner, left], axis=0)
    rpi = np.concatenate([top, rpi], axis=0)
    rpi = np.concatenate([left_corner, rpi], axis=1)
    rpi = rpi[:_S, :_S].astype(np.int32)
    pad = np.zeros((_PR, _PC), dtype=np.int32)
    pad[:_S, :_S] = rpi
    return np.ascontiguousarray(pad.reshape(-1))


_RPI_PERM = _build_rpi_perm()


def _sc_gather(table_flat, idx, h0):
    mesh = plsc.VectorSubcoreMesh(core_axis_name="c", subcore_axis_name="s")
    info = plsc.get_sparse_core_info()
    nc = info.num_cores

    @functools.partial(
        pl.kernel,
        mesh=mesh,
        out_type=jax.ShapeDtypeStruct((_HG, _PR, _PC), jnp.float32),
        scratch_types=[
            pltpu.VMEM((_HG * _NRD,), jnp.float32),
            pltpu.VMEM((_TRU,), jnp.int32),
            pltpu.VMEM((_TRU,), jnp.int32),
            pltpu.VMEM((8, _PC), jnp.float32),
            pltpu.VMEM((8, _PC), jnp.float32),
            pltpu.SemaphoreType.DMA,
            pltpu.SemaphoreType.DMA,
            pltpu.SemaphoreType.DMA,
            pltpu.SemaphoreType.DMA,
        ],
        compiler_params=pltpu.CompilerParams(
            use_tc_tiling_on_sc=True, needs_layout_passes=False
        ),
    )
    def sc_kernel(table_hbm, idx_hbm, out_hbm, tables_v, idx0_v, idx1_v,
                  ob0_v, ob1_v, semi0, semi1, semo0, semo1):
        wid = lax.axis_index("s") * nc + lax.axis_index("c")
        pltpu.sync_copy(
            table_hbm.at[pl.ds(h0 * _NRD, _HG * _NRD)], tables_v
        )

        def tile_row(c):
            return jnp.where(
                wid < 9,
                3 * wid + c,
                jnp.where(c < 2, 2 * wid + 9 + c, _TR - 1),
            )

        trs = [tile_row(c) for c in range(3)]
        idx_bufs = (idx0_v, idx1_v)
        out_bufs = (ob0_v, ob1_v)
        idx_sems = (semi0, semi1)
        out_sems = (semo0, semo1)

        def start_idx(c):
            return pltpu.async_copy(
                idx_hbm.at[pl.ds(trs[c] * _TRU, _TRU)],
                idx_bufs[c % 2],
                idx_sems[c % 2],
            )

        idx_cp = {0: start_idx(0)}
        out_cp = {}
        u = 0
        for c in range(3):
            if c + 1 < 3:
                idx_cp[c + 1] = start_idx(c + 1)
            idx_cp[c].wait()
            iv_ref = idx_bufs[c % 2]
            for h in range(_HG):
                p = u % 2
                if u >= 2:
                    out_cp[u - 2].wait()
                ob = out_bufs[p]
                hoff = h * _NRD

                @plsc.parallel_loop(0, _TRU // 16, step=1, unroll=4)
                def gbody(i, iv_ref=iv_ref, ob=ob, hoff=hoff):
                    r = i // (_PC // 16)
                    s = i % (_PC // 16)
                    iv = iv_ref[pl.ds(i * 16, 16)]
                    ob[r, pl.ds(s * 16, 16)] = plsc.load_gather(
                        tables_v, [iv + hoff]
                    )

                out_cp[u] = pltpu.async_copy(
                    ob,
                    out_hbm.at[h, pl.ds(trs[c] * 8, 8), :],
                    out_sems[p],
                )
                u += 1
        out_cp[u - 2].wait()
        out_cp[u - 1].wait()

    return sc_kernel(table_flat, idx)


def _add_body0(x_ref, p_ref, o_ref):
    o_ref[0, 0] = x_ref[0, 0] + p_ref[0, :_S, :_S]


def _add_body(prev_ref, x_ref, p_ref, o_ref):
    del prev_ref
    o_ref[0, 0] = x_ref[0, 0] + p_ref[0, :_S, :_S]


def _tc_add_stage(prev, inputs, pos, h0):
    batch = inputs.shape[0]
    out_shape = jax.ShapeDtypeStruct(inputs.shape, jnp.float32)
    x_spec = pl.BlockSpec((1, 1, _S, _S), lambda h, b: (b, h0 + h, 0, 0))
    p_spec = pl.BlockSpec((1, _PR, _PC), lambda h, b: (h, 0, 0))
    o_spec = pl.BlockSpec((1, 1, _S, _S), lambda h, b: (b, h0 + h, 0, 0))
    if prev is None:
        return pl.pallas_call(
            _add_body0,
            grid=(_HG, batch),
            in_specs=[x_spec, p_spec],
            out_specs=o_spec,
            out_shape=out_shape,
        )(inputs, pos)
    return pl.pallas_call(
        _add_body,
        grid=(_HG, batch),
        in_specs=[pl.BlockSpec(memory_space=pl.ANY), x_spec, p_spec],
        out_specs=o_spec,
        out_shape=out_shape,
        input_output_aliases={0: 0},
    )(prev, inputs, pos)


def kernel(inputs, positional_embedding):
    idx = jnp.asarray(_RPI_PERM)
    table_flat = positional_embedding.reshape(-1)
    pos_groups = [_sc_gather(table_flat, idx, g * _HG) for g in range(_NG)]
    out = None
    for g in range(_NG):
        out = _tc_add_stage(out, inputs, pos_groups[g], g * _HG)
    return out


# speedup vs baseline: 1.0393x; 1.0393x over previous
import functools

import numpy as np
import jax
import jax.numpy as jnp
from jax import lax
from jax.experimental import pallas as pl
from jax.experimental.pallas import tpu as pltpu
from jax.experimental.pallas import tpu_sc as plsc

_NUM_HEADS = 16
_ATTN_HEIGHT = 24
_CLS_TOKEN_LEN = 1
_CLS_TOKEN_POS_LEN = 3

_HG = 8
_NG = 2
_S = 577
_PR = 592
_PC = 640
_TR = 74
_TRU = 8 * _PC
_NRD = 2212


def _build_rpi_perm():
    height = _ATTN_HEIGHT
    width = (_S - _CLS_TOKEN_LEN) // height
    hh, ww = np.meshgrid(range(height), range(width))
    coords = np.stack([hh, ww], axis=-1)
    coords_flatten = np.reshape(coords, [-1, 2])
    relative_coords = coords_flatten[:, None, :] - coords_flatten[None, :, :]
    relative_coords_hh = relative_coords[:, :, 0] + height - 1
    relative_coords_ww = (relative_coords[:, :, 1] + width - 1) * (2 * height - 1)
    relative_coords = np.stack([relative_coords_hh, relative_coords_ww], axis=-1)
    rpi = np.sum(relative_coords, axis=-1).astype(np.int64)
    num_relative_distance = (2 * height - 1) * (2 * width - 1) + _CLS_TOKEN_POS_LEN
    top = np.full((1, rpi.shape[1]), num_relative_distance - 3, dtype=rpi.dtype)
    left = np.full((rpi.shape[0], 1), num_relative_distance - 2, dtype=rpi.dtype)
    corner = np.full((1, 1), num_relative_distance - 1, dtype=rpi.dtype)
    left_corner = np.concatenate([corner, left], axis=0)
    rpi = np.concatenate([top, rpi], axis=0)
    rpi = np.concatenate([left_corner, rpi], axis=1)
    rpi = rpi[:_S, :_S].astype(np.int32)
    pad = np.zeros((_PR, _PC), dtype=np.int32)
    pad[:_S, :_S] = rpi
    return np.ascontiguousarray(pad.reshape(-1))


_RPI_PERM = _build_rpi_perm()


def _sc_gather(table_flat, idx, h0):
    mesh = plsc.VectorSubcoreMesh(core_axis_name="c", subcore_axis_name="s")
    info = plsc.get_sparse_core_info()
    nc = info.num_cores

    @functools.partial(
        pl.kernel,
        mesh=mesh,
        out_type=jax.ShapeDtypeStruct((_HG, _PR, _PC), jnp.float32),
        scratch_types=[
            pltpu.VMEM((_HG * _NRD,), jnp.float32),
            pltpu.VMEM((_TRU,), jnp.int32),
            pltpu.VMEM((_TRU,), jnp.int32),
            pltpu.VMEM((8, _PC), jnp.float32),
            pltpu.VMEM((8, _PC), jnp.float32),
            pltpu.SemaphoreType.DMA,
            pltpu.SemaphoreType.DMA,
            pltpu.SemaphoreType.DMA,
            pltpu.SemaphoreType.DMA,
        ],
        compiler_params=pltpu.CompilerParams(
            use_tc_tiling_on_sc=True, needs_layout_passes=False
        ),
    )
    def sc_kernel(table_hbm, idx_hbm, out_hbm, tables_v, idx0_v, idx1_v,
                  ob0_v, ob1_v, semi0, semi1, semo0, semo1):
        wid = lax.axis_index("s") * nc + lax.axis_index("c")
        pltpu.sync_copy(
            table_hbm.at[pl.ds(h0 * _NRD, _HG * _NRD)], tables_v
        )

        def tile_row(c):
            return jnp.where(
                wid < 9,
                3 * wid + c,
                jnp.where(c < 2, 2 * wid + 9 + c, _TR - 1),
            )

        trs = [tile_row(c) for c in range(3)]
        idx_bufs = (idx0_v, idx1_v)
        out_bufs = (ob0_v, ob1_v)
        idx_sems = (semi0, semi1)
        out_sems = (semo0, semo1)

        def start_idx(c):
            return pltpu.async_copy(
                idx_hbm.at[pl.ds(trs[c] * _TRU, _TRU)],
                idx_bufs[c % 2],
                idx_sems[c % 2],
            )

        idx_cp = {0: start_idx(0)}
        out_cp = {}
        u = 0
        for c in range(3):
            if c + 1 < 3:
                idx_cp[c + 1] = start_idx(c + 1)
            idx_cp[c].wait()
            iv_ref = idx_bufs[c % 2]
            for h in range(_HG):
                p = u % 2
                if u >= 2:
                    out_cp[u - 2].wait()
                ob = out_bufs[p]
                hoff = h * _NRD

                @plsc.parallel_loop(0, _TRU // 16, step=1, unroll=4)
                def gbody(i, iv_ref=iv_ref, ob=ob, hoff=hoff):
                    r = i // (_PC // 16)
                    s = i % (_PC // 16)
                    iv = iv_ref[pl.ds(i * 16, 16)]
                    ob[r, pl.ds(s * 16, 16)] = plsc.load_gather(
                        tables_v, [iv + hoff]
                    )

                out_cp[u] = pltpu.async_copy(
                    ob,
                    out_hbm.at[h, pl.ds(trs[c] * 8, 8), :],
                    out_sems[p],
                )
                u += 1
        out_cp[u - 2].wait()
        out_cp[u - 1].wait()

    return sc_kernel(table_flat, idx)


def _add_body0(x_ref, p_ref, o_ref):
    o_ref[0, 0] = x_ref[0, 0] + p_ref[0, :_S, :_S]


def _add_body(prev_ref, x_ref, p_ref, o_ref):
    del prev_ref
    o_ref[0, 0] = x_ref[0, 0] + p_ref[0, :_S, :_S]


def _tc_add_stage(prev, inputs, pos, h0):
    batch = inputs.shape[0]
    out_shape = jax.ShapeDtypeStruct(inputs.shape, jnp.float32)
    x_spec = pl.BlockSpec((1, 1, _S, _S), lambda h, b: (b, h0 + h, 0, 0))
    p_spec = pl.BlockSpec((1, _PR, _PC), lambda h, b: (h, 0, 0))
    o_spec = pl.BlockSpec((1, 1, _S, _S), lambda h, b: (b, h0 + h, 0, 0))
    if prev is None:
        return pl.pallas_call(
            _add_body0,
            grid=(_HG, batch),
            in_specs=[x_spec, p_spec],
            out_specs=o_spec,
            out_shape=out_shape,
        )(inputs, pos)
    return pl.pallas_call(
        _add_body,
        grid=(_HG, batch),
        in_specs=[pl.BlockSpec(memory_space=pl.ANY), x_spec, p_spec],
        out_specs=o_spec,
        out_shape=out_shape,
        input_output_aliases={0: 0},
    )(prev, inputs, pos)


def kernel(inputs, positional_embedding):
    idx = jnp.asarray(_RPI_PERM)
    table_flat = positional_embedding.reshape(-1)
    pos_groups = [_sc_gather(table_flat, idx, g * _HG) for g in range(_NG)]
    out = None
    for g in range(_NG):
        out = _tc_add_stage(out, inputs, pos_groups[g], g * _HG)
    return out


# speedup vs baseline: 1.0838x; 1.0428x over previous
import functools

import numpy as np
import jax
import jax.numpy as jnp
from jax import lax
from jax.experimental import pallas as pl
from jax.experimental.pallas import tpu as pltpu
from jax.experimental.pallas import tpu_sc as plsc

_NUM_HEADS = 16
_ATTN_HEIGHT = 24
_CLS_TOKEN_LEN = 1
_CLS_TOKEN_POS_LEN = 3

_HG = 16
_NG = 1
_S = 577
_PR = 592
_PC = 640
_TR = 74
_TRU = 8 * _PC
_NRD = 2212


def _build_rpi_perm():
    height = _ATTN_HEIGHT
    width = (_S - _CLS_TOKEN_LEN) // height
    hh, ww = np.meshgrid(range(height), range(width))
    coords = np.stack([hh, ww], axis=-1)
    coords_flatten = np.reshape(coords, [-1, 2])
    relative_coords = coords_flatten[:, None, :] - coords_flatten[None, :, :]
    relative_coords_hh = relative_coords[:, :, 0] + height - 1
    relative_coords_ww = (relative_coords[:, :, 1] + width - 1) * (2 * height - 1)
    relative_coords = np.stack([relative_coords_hh, relative_coords_ww], axis=-1)
    rpi = np.sum(relative_coords, axis=-1).astype(np.int64)
    num_relative_distance = (2 * height - 1) * (2 * width - 1) + _CLS_TOKEN_POS_LEN
    top = np.full((1, rpi.shape[1]), num_relative_distance - 3, dtype=rpi.dtype)
    left = np.full((rpi.shape[0], 1), num_relative_distance - 2, dtype=rpi.dtype)
    corner = np.full((1, 1), num_relative_distance - 1, dtype=rpi.dtype)
    left_corner = np.concatenate([corner, left], axis=0)
    rpi = np.concatenate([top, rpi], axis=0)
    rpi = np.concatenate([left_corner, rpi], axis=1)
    rpi = rpi[:_S, :_S].astype(np.int32)
    pad = np.zeros((_PR, _PC), dtype=np.int32)
    pad[:_S, :_S] = rpi
    return np.ascontiguousarray(pad.reshape(-1))


_RPI_PERM = _build_rpi_perm()


def _sc_gather(table_flat, idx, h0):
    mesh = plsc.VectorSubcoreMesh(core_axis_name="c", subcore_axis_name="s")
    info = plsc.get_sparse_core_info()
    nc = info.num_cores

    @functools.partial(
        pl.kernel,
        mesh=mesh,
        out_type=jax.ShapeDtypeStruct((_HG, _PR, _PC), jnp.float32),
        scratch_types=[
            pltpu.VMEM((_HG * _NRD,), jnp.float32),
            pltpu.VMEM((3 * _TRU,), jnp.int32),
            pltpu.VMEM((24, _PC), jnp.float32),
            pltpu.VMEM((24, _PC), jnp.float32),
            pltpu.SemaphoreType.DMA,
            pltpu.SemaphoreType.DMA,
            pltpu.SemaphoreType.DMA,
        ],
        compiler_params=pltpu.CompilerParams(
            use_tc_tiling_on_sc=True, needs_layout_passes=False
        ),
    )
    def sc_kernel(table_hbm, idx_hbm, out_hbm, tables_v, idx_v,
                  ob0_v, ob1_v, semi0, semo0, semo1):
        wid = lax.axis_index("s") * nc + lax.axis_index("c")
        pltpu.sync_copy(
            table_hbm.at[pl.ds(h0 * _NRD, _HG * _NRD)], tables_v
        )

        trA = jnp.where(wid < 9, 3 * wid, 2 * wid + 9)
        trB = jnp.where(wid < 9, 3 * wid + 2, _TR - 1)

        cpA = pltpu.async_copy(
            idx_hbm.at[pl.ds(trA * _TRU, 2 * _TRU)],
            idx_v.at[pl.ds(0, 2 * _TRU)],
            semi0,
        )
        cpB = pltpu.async_copy(
            idx_hbm.at[pl.ds(trB * _TRU, _TRU)],
            idx_v.at[pl.ds(2 * _TRU, _TRU)],
            semi0,
        )
        cpA.wait()
        cpB.wait()

        out_bufs = (ob0_v, ob1_v)
        out_sems = (semo0, semo1)
        out_cp = {}
        for h in range(_HG):
            p = h % 2
            if h >= 2:
                for cp in out_cp[h - 2]:
                    cp.wait()
            ob = out_bufs[p]
            hoff = h * _NRD

            @plsc.parallel_loop(0, 3 * _TRU // 16, step=1, unroll=4)
            def gbody(i, ob=ob, hoff=hoff):
                r = i // (_PC // 16)
                s = i % (_PC // 16)
                iv = idx_v[pl.ds(i * 16, 16)]
                ob[r, pl.ds(s * 16, 16)] = plsc.load_gather(
                    tables_v, [iv + hoff]
                )

            out_cp[h] = [
                pltpu.async_copy(
                    ob.at[pl.ds(0, 16), :],
                    out_hbm.at[h, pl.ds(trA * 8, 16), :],
                    out_sems[p],
                ),
                pltpu.async_copy(
                    ob.at[pl.ds(16, 8), :],
                    out_hbm.at[h, pl.ds(trB * 8, 8), :],
                    out_sems[p],
                ),
            ]
        for h in (_HG - 2, _HG - 1):
            for cp in out_cp[h]:
                cp.wait()

    return sc_kernel(table_flat, idx)


def _add_body0(x_ref, p_ref, o_ref):
    o_ref[0, 0] = x_ref[0, 0] + p_ref[0, :_S, :_S]


def _add_body(prev_ref, x_ref, p_ref, o_ref):
    del prev_ref
    o_ref[0, 0] = x_ref[0, 0] + p_ref[0, :_S, :_S]


def _tc_add_stage(prev, inputs, pos, h0):
    batch = inputs.shape[0]
    out_shape = jax.ShapeDtypeStruct(inputs.shape, jnp.float32)
    x_spec = pl.BlockSpec((1, 1, _S, _S), lambda h, b: (b, h0 + h, 0, 0))
    p_spec = pl.BlockSpec((1, _PR, _PC), lambda h, b: (h, 0, 0))
    o_spec = pl.BlockSpec((1, 1, _S, _S), lambda h, b: (b, h0 + h, 0, 0))
    if prev is None:
        return pl.pallas_call(
            _add_body0,
            grid=(_HG, batch),
            in_specs=[x_spec, p_spec],
            out_specs=o_spec,
            out_shape=out_shape,
        )(inputs, pos)
    return pl.pallas_call(
        _add_body,
        grid=(_HG, batch),
        in_specs=[pl.BlockSpec(memory_space=pl.ANY), x_spec, p_spec],
        out_specs=o_spec,
        out_shape=out_shape,
        input_output_aliases={0: 0},
    )(prev, inputs, pos)


def kernel(inputs, positional_embedding):
    idx = jnp.asarray(_RPI_PERM)
    table_flat = positional_embedding.reshape(-1)
    pos_groups = [_sc_gather(table_flat, idx, g * _HG) for g in range(_NG)]
    out = None
    for g in range(_NG):
        out = _tc_add_stage(out, inputs, pos_groups[g], g * _HG)
    return out


# speedup vs baseline: 1.0889x; 1.0047x over previous
import functools

import numpy as np
import jax
import jax.numpy as jnp
from jax import lax
from jax.experimental import pallas as pl
from jax.experimental.pallas import tpu as pltpu
from jax.experimental.pallas import tpu_sc as plsc

_NUM_HEADS = 16
_ATTN_HEIGHT = 24
_CLS_TOKEN_LEN = 1
_CLS_TOKEN_POS_LEN = 3

_HG = 16
_NG = 1
_S = 577
_PR = 592
_PC = 640
_TR = 74
_TRU = 8 * _PC
_NRD = 2212


def _build_rpi_perm():
    height = _ATTN_HEIGHT
    width = (_S - _CLS_TOKEN_LEN) // height
    hh, ww = np.meshgrid(range(height), range(width))
    coords = np.stack([hh, ww], axis=-1)
    coords_flatten = np.reshape(coords, [-1, 2])
    relative_coords = coords_flatten[:, None, :] - coords_flatten[None, :, :]
    relative_coords_hh = relative_coords[:, :, 0] + height - 1
    relative_coords_ww = (relative_coords[:, :, 1] + width - 1) * (2 * height - 1)
    relative_coords = np.stack([relative_coords_hh, relative_coords_ww], axis=-1)
    rpi = np.sum(relative_coords, axis=-1).astype(np.int64)
    num_relative_distance = (2 * height - 1) * (2 * width - 1) + _CLS_TOKEN_POS_LEN
    top = np.full((1, rpi.shape[1]), num_relative_distance - 3, dtype=rpi.dtype)
    left = np.full((rpi.shape[0], 1), num_relative_distance - 2, dtype=rpi.dtype)
    corner = np.full((1, 1), num_relative_distance - 1, dtype=rpi.dtype)
    left_corner = np.concatenate([corner, left], axis=0)
    rpi = np.concatenate([top, rpi], axis=0)
    rpi = np.concatenate([left_corner, rpi], axis=1)
    rpi = rpi[:_S, :_S].astype(np.int32)
    pad = np.zeros((_PR, _PC), dtype=np.int32)
    pad[:_S, :_S] = rpi
    return np.ascontiguousarray(pad.reshape(-1))


_RPI_PERM = _build_rpi_perm()


def _sc_gather(table_flat, idx, h0):
    mesh = plsc.VectorSubcoreMesh(core_axis_name="c", subcore_axis_name="s")
    info = plsc.get_sparse_core_info()
    nc = info.num_cores

    @functools.partial(
        pl.kernel,
        mesh=mesh,
        out_type=jax.ShapeDtypeStruct((_HG, _PR, _PC), jnp.float32),
        scratch_types=[
            pltpu.VMEM((_HG * _NRD,), jnp.float32),
            pltpu.VMEM((3 * _TRU,), jnp.int32),
            pltpu.VMEM((24, _PC), jnp.float32),
            pltpu.VMEM((24, _PC), jnp.float32),
            pltpu.VMEM((24, _PC), jnp.float32),
            pltpu.VMEM((24, _PC), jnp.float32),
            pltpu.SemaphoreType.DMA,
            pltpu.SemaphoreType.DMA,
            pltpu.SemaphoreType.DMA,
        ],
        compiler_params=pltpu.CompilerParams(
            use_tc_tiling_on_sc=True, needs_layout_passes=False
        ),
    )
    def sc_kernel(table_hbm, idx_hbm, out_hbm, tables_v, idx_v,
                  ob0a_v, ob0b_v, ob1a_v, ob1b_v, semi0, semo0, semo1):
        wid = lax.axis_index("s") * nc + lax.axis_index("c")
        pltpu.sync_copy(
            table_hbm.at[pl.ds(h0 * _NRD, _HG * _NRD)], tables_v
        )

        trA = jnp.where(wid < 9, 3 * wid, 2 * wid + 9)
        trB = jnp.where(wid < 9, 3 * wid + 2, _TR - 1)

        cpA = pltpu.async_copy(
            idx_hbm.at[pl.ds(trA * _TRU, 2 * _TRU)],
            idx_v.at[pl.ds(0, 2 * _TRU)],
            semi0,
        )
        cpB = pltpu.async_copy(
            idx_hbm.at[pl.ds(trB * _TRU, _TRU)],
            idx_v.at[pl.ds(2 * _TRU, _TRU)],
            semi0,
        )
        cpA.wait()
        cpB.wait()

        out_bufs = ((ob0a_v, ob0b_v), (ob1a_v, ob1b_v))
        out_sems = (semo0, semo1)
        out_cp = {}
        npairs = _HG // 2
        for hp in range(npairs):
            p = hp % 2
            if hp >= 2:
                for cp in out_cp[hp - 2]:
                    cp.wait()
            ob_a, ob_b = out_bufs[p]
            off_a = (2 * hp) * _NRD
            off_b = (2 * hp + 1) * _NRD

            def rbody(r, carry, ob_a=ob_a, ob_b=ob_b, off_a=off_a, off_b=off_b):
                rb = r * _PC

                @plsc.parallel_loop(0, _PC // 16, step=1, unroll=4)
                def sbody(s, ob_a=ob_a, ob_b=ob_b, rb=rb, r=r):
                    iv = idx_v[pl.ds(rb + s * 16, 16)]
                    ob_a[r, pl.ds(s * 16, 16)] = plsc.load_gather(
                        tables_v, [iv + off_a]
                    )
                    ob_b[r, pl.ds(s * 16, 16)] = plsc.load_gather(
                        tables_v, [iv + off_b]
                    )

                return carry

            lax.fori_loop(0, 24, rbody, 0)

            cps = []
            for ob, h in ((ob_a, 2 * hp), (ob_b, 2 * hp + 1)):
                cps.append(pltpu.async_copy(
                    ob.at[pl.ds(0, 16), :],
                    out_hbm.at[h, pl.ds(trA * 8, 16), :],
                    out_sems[p],
                ))
                cps.append(pltpu.async_copy(
                    ob.at[pl.ds(16, 8), :],
                    out_hbm.at[h, pl.ds(trB * 8, 8), :],
                    out_sems[p],
                ))
            out_cp[hp] = cps
        for hp in (npairs - 2, npairs - 1):
            for cp in out_cp[hp]:
                cp.wait()

    return sc_kernel(table_flat, idx)


def _add_body0(x_ref, p_ref, o_ref):
    o_ref[0, 0] = x_ref[0, 0] + p_ref[0, :_S, :_S]


def _add_body(prev_ref, x_ref, p_ref, o_ref):
    del prev_ref
    o_ref[0, 0] = x_ref[0, 0] + p_ref[0, :_S, :_S]


def _tc_add_stage(prev, inputs, pos, h0):
    batch = inputs.shape[0]
    out_shape = jax.ShapeDtypeStruct(inputs.shape, jnp.float32)
    x_spec = pl.BlockSpec((1, 1, _S, _S), lambda h, b: (b, h0 + h, 0, 0))
    p_spec = pl.BlockSpec((1, _PR, _PC), lambda h, b: (h, 0, 0))
    o_spec = pl.BlockSpec((1, 1, _S, _S), lambda h, b: (b, h0 + h, 0, 0))
    if prev is None:
        return pl.pallas_call(
            _add_body0,
            grid=(_HG, batch),
            in_specs=[x_spec, p_spec],
            out_specs=o_spec,
            out_shape=out_shape,
        )(inputs, pos)
    return pl.pallas_call(
        _add_body,
        grid=(_HG, batch),
        in_specs=[pl.BlockSpec(memory_space=pl.ANY), x_spec, p_spec],
        out_specs=o_spec,
        out_shape=out_shape,
        input_output_aliases={0: 0},
    )(prev, inputs, pos)


def kernel(inputs, positional_embedding):
    idx = jnp.asarray(_RPI_PERM)
    table_flat = positional_embedding.reshape(-1)
    pos_groups = [_sc_gather(table_flat, idx, g * _HG) for g in range(_NG)]
    out = None
    for g in range(_NG):
        out = _tc_add_stage(out, inputs, pos_groups[g], g * _HG)
    return out
